# baseline (speedup 1.0000x reference)
import functools

import jax
import jax.numpy as jnp
from jax import lax
from jax.experimental import pallas as pl
from jax.experimental.pallas import tpu as pltpu
from jax.experimental.pallas import tpu_sc as plsc

B = 16384
EMB = 64
NV = 1000000
NC = 2
NS = 16
NW = NC * NS
BPW = B // NW
CHUNK = 128
NCHUNK = BPW // CHUNK


def _sc_gather_t(uid2d, iid2d, utt, itt):
    mesh = plsc.VectorSubcoreMesh(core_axis_name="c", subcore_axis_name="s")

    @functools.partial(
        pl.kernel,
        out_type=[
            jax.ShapeDtypeStruct((EMB, B), jnp.float32),
            jax.ShapeDtypeStruct((EMB, B), jnp.float32),
        ],
        mesh=mesh,
        scratch_types=[
            pltpu.VMEM((NCHUNK, CHUNK), jnp.int32),
            pltpu.VMEM((EMB, BPW), jnp.float32),
            pltpu.SemaphoreType.DMA,
        ],
        compiler_params=pltpu.CompilerParams(use_tc_tiling_on_sc=False),
    )
    def k(uid_hbm, iid_hbm, ut_hbm, it_hbm, uout, iout, ids, buf, sem):
        wid = lax.axis_index("s") * NC + lax.axis_index("c")
        base = wid * BPW
        rowbase = wid * NCHUNK

        for src_ids, table, out in ((uid_hbm, ut_hbm, uout),
                                    (iid_hbm, it_hbm, iout)):
            pltpu.sync_copy(src_ids.at[pl.ds(rowbase, NCHUNK)], ids)

            def body(c, _, table=table):
                copies = []
                for d in range(EMB):
                    copies.append(pltpu.async_copy(
                        table.at[d].at[ids.at[c]],
                        buf.at[d, pl.ds(c * CHUNK, CHUNK)], sem))
                for cp in copies:
                    cp.wait()
                return 0

            lax.fori_loop(0, NCHUNK, body, 0)
            pltpu.sync_copy(buf, out.at[:, pl.ds(base, BPW)])

    return k(uid2d, iid2d, utt, itt)


def _mlp_body(u_ref, i_ref, w1_ref, b1_ref, w2_ref, b2_ref,
              w3_ref, b3_ref, w4_ref, b4_ref, o_ref):
    dn = (((1,), (0,)), ((), ()))
    w1 = w1_ref[...]
    h = (lax.dot_general(w1[:, :EMB], u_ref[...], dn,
                         preferred_element_type=jnp.float32)
         + lax.dot_general(w1[:, EMB:], i_ref[...], dn,
                           preferred_element_type=jnp.float32))
    h = jnp.maximum(h + b1_ref[...], 0.0)
    h = lax.dot_general(w2_ref[...], h, dn, preferred_element_type=jnp.float32)
    h = jnp.maximum(h + b2_ref[...], 0.0)
    h = lax.dot_general(w3_ref[...], h, dn, preferred_element_type=jnp.float32)
    h = jnp.maximum(h + b3_ref[...], 0.0)
    o = lax.dot_general(w4_ref[...], h, dn, preferred_element_type=jnp.float32)
    o_ref[...] = o + b4_ref[...]


def _tc_mlp(uxt, ixt, W1, b1, W2, b2, W3, b3, W4, b4, blk=2048):
    grid = (B // blk,)
    full = lambda b: (0, 0)
    return pl.pallas_call(
        _mlp_body,
        grid=grid,
        in_specs=[
            pl.BlockSpec((EMB, blk), lambda b: (0, b)),
            pl.BlockSpec((EMB, blk), lambda b: (0, b)),
            pl.BlockSpec(W1.shape, full),
            pl.BlockSpec((256, 1), full),
            pl.BlockSpec(W2.shape, full),
            pl.BlockSpec((128, 1), full),
            pl.BlockSpec(W3.shape, full),
            pl.BlockSpec((64, 1), full),
            pl.BlockSpec(W4.shape, full),
            pl.BlockSpec((1, 1), full),
        ],
        out_specs=pl.BlockSpec((1, blk), lambda b: (0, b)),
        out_shape=jax.ShapeDtypeStruct((1, B), jnp.float32),
    )(uxt, ixt, W1, b1.reshape(256, 1), W2, b2.reshape(128, 1),
      W3, b3.reshape(64, 1), W4, b4.reshape(1, 1))


def kernel(user_ids, item_ids, user_table, item_table,
           W1, b1, W2, b2, W3, b3, W4, b4):
    uid2d = user_ids.astype(jnp.int32).reshape(NW * NCHUNK, CHUNK)
    iid2d = item_ids.astype(jnp.int32).reshape(NW * NCHUNK, CHUNK)
    uxt, ixt = _sc_gather_t(uid2d, iid2d, user_table.T, item_table.T)
    out = _tc_mlp(uxt, ixt, W1, b1, W2, b2, W3, b3, W4, b4)
    return out[0]


# speedup vs baseline: 13.6866x; 13.6866x over previous
import functools

import jax
import jax.numpy as jnp
from jax import lax
from jax.experimental import pallas as pl
from jax.experimental.pallas import tpu as pltpu
from jax.experimental.pallas import tpu_sc as plsc

B = 16384
EMB = 64
NC = 2
NS = 16
NW = NC * NS
BPW = B // NW
LANES = 16
NG = BPW // LANES
BUFROWS = 256
GPB = 4
NB = BUFROWS // (GPB * LANES)


def _sc_gather(uid2d, iid2d, user_table, item_table):
    mesh = plsc.VectorSubcoreMesh(core_axis_name="c", subcore_axis_name="s")

    @functools.partial(
        pl.kernel,
        out_type=[
            jax.ShapeDtypeStruct((B, EMB), jnp.float32),
            jax.ShapeDtypeStruct((B, EMB), jnp.float32),
        ],
        mesh=mesh,
        scratch_types=[
            pltpu.VMEM((NG, LANES), jnp.int32),
            pltpu.VMEM((NG, LANES), jnp.int32),
            pltpu.VMEM((BUFROWS, EMB), jnp.float32),
            pltpu.SemaphoreType.DMA,
        ],
    )
    def k(uid_hbm, iid_hbm, ut_hbm, it_hbm, uout, iout,
          uidx, iidx, buf, sem):
        wid = lax.axis_index("s") * NC + lax.axis_index("c")
        base = wid * BPW
        rowbase = wid * NG
        pltpu.sync_copy(uid_hbm.at[pl.ds(rowbase, NG)], uidx)
        pltpu.sync_copy(iid_hbm.at[pl.ds(rowbase, NG)], iidx)

        for idx, table, out in ((uidx, ut_hbm, uout), (iidx, it_hbm, iout)):
            for h in range(BPW // BUFROWS):
                g0 = h * (BUFROWS // LANES)

                def body(b, _, idx=idx, table=table, g0=g0):
                    copies = []
                    for g in range(GPB):
                        grp = g0 + b * GPB + g
                        ids = idx[grp, pl.ds(0, LANES)]
                        for l in range(LANES):
                            dst = (b * GPB + g) * LANES + l
                            copies.append(pltpu.async_copy(
                                table.at[pl.ds(ids[l], 1)],
                                buf.at[pl.ds(dst, 1)], sem))
                    for c in copies:
                        c.wait()
                    return 0

                lax.fori_loop(0, NB, body, 0)
                pltpu.sync_copy(
                    buf, out.at[pl.ds(base + h * BUFROWS, BUFROWS)])

    return k(uid2d, iid2d, user_table, item_table)


def _mlp_body(u_ref, i_ref, w1_ref, b1_ref, w2_ref, b2_ref,
              w3_ref, b3_ref, w4_ref, b4_ref, o_ref):
    dn = (((1,), (1,)), ((), ()))
    x = jnp.concatenate([u_ref[...], i_ref[...]], axis=1)
    h = lax.dot_general(x, w1_ref[...], dn, preferred_element_type=jnp.float32)
    h = jnp.maximum(h + b1_ref[...], 0.0)
    h = lax.dot_general(h, w2_ref[...], dn, preferred_element_type=jnp.float32)
    h = jnp.maximum(h + b2_ref[...], 0.0)
    h = lax.dot_general(h, w3_ref[...], dn, preferred_element_type=jnp.float32)
    h = jnp.maximum(h + b3_ref[...], 0.0)
    o = jnp.sum(h * w4_ref[...], axis=1, keepdims=True) + b4_ref[...]
    o_ref[...] = o


def _tc_mlp(u_emb, i_emb, W1, b1, W2, b2, W3, b3, W4, b4, blk=2048):
    grid = (B // blk,)
    full = lambda b: (0, 0)
    return pl.pallas_call(
        _mlp_body,
        grid=grid,
        in_specs=[
            pl.BlockSpec((blk, EMB), lambda b: (b, 0)),
            pl.BlockSpec((blk, EMB), lambda b: (b, 0)),
            pl.BlockSpec(W1.shape, full),
            pl.BlockSpec((1, 256), full),
            pl.BlockSpec(W2.shape, full),
            pl.BlockSpec((1, 128), full),
            pl.BlockSpec(W3.shape, full),
            pl.BlockSpec((1, 64), full),
            pl.BlockSpec(W4.shape, full),
            pl.BlockSpec((1, 1), full),
        ],
        out_specs=pl.BlockSpec((blk, 1), lambda b: (b, 0)),
        out_shape=jax.ShapeDtypeStruct((B, 1), jnp.float32),
    )(u_emb, i_emb, W1, b1.reshape(1, 256), W2, b2.reshape(1, 128),
      W3, b3.reshape(1, 64), W4, b4.reshape(1, 1))


def kernel(user_ids, item_ids, user_table, item_table,
           W1, b1, W2, b2, W3, b3, W4, b4):
    uid2d = user_ids.astype(jnp.int32).reshape(NW * NG, LANES)
    iid2d = item_ids.astype(jnp.int32).reshape(NW * NG, LANES)
    u_emb, i_emb = _sc_gather(uid2d, iid2d, user_table, item_table)
    out = _tc_mlp(u_emb, i_emb, W1, b1, W2, b2, W3, b3, W4, b4)
    return out[:, 0]
